# baseline (speedup 1.0000x reference)
import jax
import jax.numpy as jnp
from jax import lax
from jax.experimental import pallas as pl
from jax.experimental.pallas import tpu as pltpu
from jax.experimental.pallas import tpu_sc as plsc

_NC, _NS, _L = 2, 16, 16


def _sc_fill(t_flat, *, b, d, h, w):
    nw = _NC * _NS
    cpw = d // nw
    plane = h * w
    nvec = w // _L
    d_half = d // 2
    mesh = plsc.VectorSubcoreMesh(core_axis_name="c", subcore_axis_name="s")

    def body(t_hbm, o_hbm, stage, buf0, buf1, sem0, sem1):
        wid = lax.axis_index("s") * _NC + lax.axis_index("c")
        c0 = wid * cpw
        pltpu.make_async_copy(
            t_hbm.at[pl.ds(c0 * w, cpw * w)], stage, sem0
        ).start()
        pltpu.make_async_copy(
            t_hbm.at[pl.ds(c0 * w, cpw * w)], stage, sem0
        ).wait()
        bufs, sems = (buf0, buf1), (sem0, sem1)

        def build_col(k, buf):
            vs = [stage[pl.ds(k * w + j * _L, _L)] for j in range(nvec)]

            def row(i, carry):
                base = i * w
                for j in range(nvec):
                    buf[pl.ds(base + j * _L, _L)] = vs[j]
                return carry

            lax.fori_loop(0, h, row, 0, unroll=False)

        def build_row(k, buf):
            def row_group(g, carry):
                v16 = stage[pl.ds(k * w + g * _L, _L)]
                for l in range(_L):
                    bc = lax.gather(
                        v16,
                        jnp.full((_L, 1), l, jnp.int32),
                        lax.GatherDimensionNumbers(
                            offset_dims=(),
                            collapsed_slice_dims=(0,),
                            start_index_map=(0,),
                        ),
                        (1,),
                        mode=lax.GatherScatterMode.PROMISE_IN_BOUNDS,
                    )
                    base = (g * _L + l) * w
                    for j in range(nvec):
                        buf[pl.ds(base + j * _L, _L)] = bc
                return carry

            lax.fori_loop(0, h // _L, row_group, 0, unroll=False)

        def run(build):
            waits = {}
            for k in range(cpw):
                buf, sem = bufs[k % 2], sems[k % 2]
                if k >= 2:
                    for cp in waits[k - 2]:
                        cp.wait()
                build(k, buf)
                waits[k] = []
                for bb in range(b):
                    cp = pltpu.make_async_copy(
                        buf, o_hbm.at[bb * d + c0 + k], sem
                    )
                    cp.start()
                    waits[k].append(cp)
            for k in (cpw - 2, cpw - 1):
                for cp in waits[k]:
                    cp.wait()

        @pl.when(c0 < d_half)
        def _col():
            run(build_col)

        @pl.when(c0 >= d_half)
        def _row():
            run(build_row)

    f = pl.kernel(
        body,
        out_type=jax.ShapeDtypeStruct((b * d, plane), jnp.float32),
        mesh=mesh,
        scratch_types=[
            pltpu.VMEM((cpw * w,), jnp.float32),
            pltpu.VMEM((plane,), jnp.float32),
            pltpu.VMEM((plane,), jnp.float32),
            pltpu.SemaphoreType.DMA,
            pltpu.SemaphoreType.DMA,
        ],
    )
    return f(t_flat)


def kernel(x, row_embed, col_embed):
    b = x.shape[0]
    h, w = x.shape[2], x.shape[3]
    d_half = row_embed.shape[1]
    d = 2 * d_half
    t = jnp.concatenate([col_embed[:w].T, row_embed[:h].T], axis=0).reshape(-1)
    out = _sc_fill(t, b=b, d=d, h=h, w=w)
    return out.reshape(b, d, h, w).astype(x.dtype)


# speedup vs baseline: 1.4927x; 1.4927x over previous
import jax
import jax.numpy as jnp
from jax import lax
from jax.experimental import pallas as pl
from jax.experimental.pallas import tpu as pltpu
from jax.experimental.pallas import tpu_sc as plsc

_NC, _NS, _L = 2, 16, 16


def _splat(v16, lane):
    return lax.gather(
        v16,
        jnp.full((_L, 1), lane, jnp.int32),
        lax.GatherDimensionNumbers(
            offset_dims=(), collapsed_slice_dims=(0,), start_index_map=(0,)
        ),
        (1,),
        mode=lax.GatherScatterMode.PROMISE_IN_BOUNDS,
    )


def _sc_fill(t_flat, *, b, d, h, w):
    nw = _NC * _NS
    cpw = d // nw
    nvec = w // _L
    d_half = d // 2
    mesh = plsc.VectorSubcoreMesh(core_axis_name="c", subcore_axis_name="s")

    def body(t_hbm, o_hbm, stage, buf0, buf1, sem0, sem1):
        wid = lax.axis_index("s") * _NC + lax.axis_index("c")
        c0 = wid * cpw
        cp_in = pltpu.make_async_copy(
            t_hbm.at[pl.ds(c0 * w, cpw * w)], stage, sem0
        )
        cp_in.start()
        cp_in.wait()
        bufs, sems = (buf0, buf1), (sem0, sem1)

        def build_col(k, buf):
            vs = [stage[pl.ds(k * w + j * _L, _L)] for j in range(nvec)]

            def row(i, carry):
                for j in range(nvec):
                    buf.at[i][pl.ds(j * _L, _L)] = vs[j]
                return carry

            lax.fori_loop(0, h, row, 0, unroll=False)

        def build_row(k, buf):
            def row_group(g, carry):
                v16 = stage[pl.ds(k * w + g * _L, _L)]
                for l in range(_L):
                    bc = _splat(v16, l)
                    i = g * _L + l
                    for j in range(nvec):
                        buf.at[i][pl.ds(j * _L, _L)] = bc
                return carry

            lax.fori_loop(0, h // _L, row_group, 0, unroll=False)

        def run(build):
            waits = {}
            for k in range(cpw):
                buf, sem = bufs[k % 2], sems[k % 2]
                if k >= 2:
                    for cp in waits[k - 2]:
                        cp.wait()
                build(k, buf)
                waits[k] = []
                for bb in range(b):
                    cp = pltpu.make_async_copy(buf, o_hbm.at[bb, c0 + k], sem)
                    cp.start()
                    waits[k].append(cp)
            for k in (cpw - 2, cpw - 1):
                for cp in waits[k]:
                    cp.wait()

        @pl.when(c0 < d_half)
        def _col():
            run(build_col)

        @pl.when(c0 >= d_half)
        def _row():
            run(build_row)

    f = pl.kernel(
        body,
        out_type=jax.ShapeDtypeStruct((b, d, h, w), jnp.float32),
        mesh=mesh,
        scratch_types=[
            pltpu.VMEM((cpw * w,), jnp.float32),
            pltpu.VMEM((h, w), jnp.float32),
            pltpu.VMEM((h, w), jnp.float32),
            pltpu.SemaphoreType.DMA,
            pltpu.SemaphoreType.DMA,
        ],
        compiler_params=pltpu.CompilerParams(use_tc_tiling_on_sc=True),
    )
    return f(t_flat)


def kernel(x, row_embed, col_embed):
    b = x.shape[0]
    h, w = x.shape[2], x.shape[3]
    d_half = row_embed.shape[1]
    d = 2 * d_half
    t = jnp.concatenate([col_embed[:w].T, row_embed[:h].T], axis=0).reshape(-1)
    return _sc_fill(t, b=b, d=d, h=h, w=w).astype(x.dtype)


# speedup vs baseline: 4.7190x; 3.1615x over previous
import jax
import jax.numpy as jnp
from jax import lax
from jax.experimental import pallas as pl
from jax.experimental.pallas import tpu as pltpu
from jax.experimental.pallas import tpu_sc as plsc

_NC, _NS, _L = 2, 16, 16
_CHUNK = 32


def _sc_fill(t_hbm_shape, *, b, d, h, w):
    nw = _NC * _NS
    nblk = b * h // nw
    d_half = d // 2
    nv = d_half // _L
    mesh = plsc.VectorSubcoreMesh(core_axis_name="c", subcore_axis_name="s")

    def body(t_hbm, o_hbm, scol, srow, buf0, buf1, sem0, sem1):
        wid = lax.axis_index("s") * _NC + lax.axis_index("c")
        q0 = wid * nblk
        b0 = q0 // h
        i0 = q0 % h
        bufs, sems = (buf0, buf1), (sem0, sem1)

        i0m = i0 % 8
        cp = pltpu.make_async_copy(
            t_hbm.at[pl.ds(pl.multiple_of(w + i0 - i0m, 8), _CHUNK)], srow, sem0
        )
        cp.start()
        cp.wait()

        for c in range(w // _CHUNK):
            cp = pltpu.make_async_copy(
                t_hbm.at[pl.ds(c * _CHUNK, _CHUNK)], scol, sem0
            )
            cp.start()
            cp.wait()

            def colrow(r, carry):
                j = c * _CHUNK + r
                for jj in range(nv):
                    v = scol.at[r][pl.ds(jj * _L, _L)]
                    buf0.at[j][pl.ds(jj * _L, _L)] = v
                    buf1.at[j][pl.ds(jj * _L, _L)] = v
                return carry

            lax.fori_loop(0, _CHUNK, colrow, 0, unroll=False)

        waits = {}
        for k in range(nblk):
            buf, sem = bufs[k % 2], sems[k % 2]
            if k >= 2:
                waits[k - 2].wait()
            vs = [srow.at[i0m + k][pl.ds(jj * _L, _L)] for jj in range(nv)]

            def rowfill(j, carry):
                for jj in range(nv):
                    buf.at[j][pl.ds(d_half + jj * _L, _L)] = vs[jj]
                return carry

            lax.fori_loop(0, h, rowfill, 0, unroll=False)
            cp = pltpu.make_async_copy(buf, o_hbm.at[b0, i0 + k], sem)
            cp.start()
            waits[k] = cp
        waits[nblk - 2].wait()
        waits[nblk - 1].wait()

    return pl.kernel(
        body,
        out_type=jax.ShapeDtypeStruct((b, h, w, d), jnp.float32),
        mesh=mesh,
        scratch_types=[
            pltpu.VMEM((_CHUNK, d_half), jnp.float32),
            pltpu.VMEM((_CHUNK, d_half), jnp.float32),
            pltpu.VMEM((w, d), jnp.float32),
            pltpu.VMEM((w, d), jnp.float32),
            pltpu.SemaphoreType.DMA,
            pltpu.SemaphoreType.DMA,
        ],
        compiler_params=pltpu.CompilerParams(use_tc_tiling_on_sc=True),
    )


def kernel(x, row_embed, col_embed):
    b = x.shape[0]
    h, w = x.shape[2], x.shape[3]
    d_half = row_embed.shape[1]
    d = 2 * d_half
    t = jnp.concatenate([col_embed[:w], row_embed[:h]], axis=0)
    out_bhwd = _sc_fill(t.shape, b=b, d=d, h=h, w=w)(t)
    return jnp.transpose(out_bhwd, (0, 3, 1, 2)).astype(x.dtype)


# speedup vs baseline: 4.7836x; 1.0137x over previous
import jax
import jax.numpy as jnp
from jax import lax
from jax.experimental import pallas as pl
from jax.experimental.pallas import tpu as pltpu
from jax.experimental.pallas import tpu_sc as plsc

_NC, _NS, _L = 2, 16, 16
_CHUNK = 32


def _sc_fill(t_hbm_shape, *, b, d, h, w):
    nw = _NC * _NS
    nblk = b * h // nw
    d_half = d // 2
    nv = d_half // _L
    mesh = plsc.VectorSubcoreMesh(core_axis_name="c", subcore_axis_name="s")

    def body(t_hbm, o_hbm, scol, srow, buf0, buf1, sem0, sem1):
        wid = lax.axis_index("s") * _NC + lax.axis_index("c")
        ipw = h // nw
        i0 = wid * ipw
        bufs, sems = (buf0, buf1), (sem0, sem1)

        i0m = i0 % 8
        cp = pltpu.make_async_copy(
            t_hbm.at[pl.ds(pl.multiple_of(w + i0 - i0m, 8), 16)], srow, sem0
        )
        cp.start()
        cp.wait()

        for c in range(w // _CHUNK):
            cp = pltpu.make_async_copy(
                t_hbm.at[pl.ds(c * _CHUNK, _CHUNK)], scol, sem0
            )
            cp.start()
            cp.wait()

            def colrow(r, carry):
                j = c * _CHUNK + r
                for jj in range(nv):
                    v = scol.at[r][pl.ds(jj * _L, _L)]
                    buf0.at[j][pl.ds(jj * _L, _L)] = v
                    buf1.at[j][pl.ds(jj * _L, _L)] = v
                return carry

            lax.fori_loop(0, _CHUNK, colrow, 0, unroll=False)

        waits = {}
        for k in range(ipw):
            buf, sem = bufs[k % 2], sems[k % 2]
            if k >= 2:
                for cp in waits[k - 2]:
                    cp.wait()
            vs = [srow.at[i0m + k][pl.ds(jj * _L, _L)] for jj in range(nv)]

            def rowfill(j, carry):
                for jj in range(nv):
                    buf.at[j][pl.ds(d_half + jj * _L, _L)] = vs[jj]
                return carry

            lax.fori_loop(0, h, rowfill, 0, unroll=False)
            waits[k] = []
            for bb in range(b):
                cp = pltpu.make_async_copy(buf, o_hbm.at[bb, i0 + k], sem)
                cp.start()
                waits[k].append(cp)
        for k in (ipw - 2, ipw - 1):
            for cp in waits[k]:
                cp.wait()

    return pl.kernel(
        body,
        out_type=jax.ShapeDtypeStruct((b, h, w, d), jnp.float32),
        mesh=mesh,
        scratch_types=[
            pltpu.VMEM((_CHUNK, d_half), jnp.float32),
            pltpu.VMEM((16, d_half), jnp.float32),
            pltpu.VMEM((w, d), jnp.float32),
            pltpu.VMEM((w, d), jnp.float32),
            pltpu.SemaphoreType.DMA,
            pltpu.SemaphoreType.DMA,
        ],
        compiler_params=pltpu.CompilerParams(use_tc_tiling_on_sc=True),
    )


def kernel(x, row_embed, col_embed):
    b = x.shape[0]
    h, w = x.shape[2], x.shape[3]
    d_half = row_embed.shape[1]
    d = 2 * d_half
    t = jnp.concatenate(
        [col_embed[:w], row_embed[:h], jnp.zeros((8, d_half), jnp.float32)],
        axis=0,
    )
    out_bhwd = _sc_fill(t.shape, b=b, d=d, h=h, w=w)(t)
    return jnp.transpose(out_bhwd, (0, 3, 1, 2)).astype(x.dtype)


# speedup vs baseline: 4.8768x; 1.0195x over previous
import jax
import jax.numpy as jnp
from jax import lax
from jax.experimental import pallas as pl
from jax.experimental.pallas import tpu as pltpu
from jax.experimental.pallas import tpu_sc as plsc

_NC, _NS, _L = 2, 16, 16
_CHUNK = 56


def _sc_fill(*, b, d, h, w):
    nw = _NC * _NS
    ipw = h // nw
    d_half = d // 2
    nv = d_half // _L
    mesh = plsc.VectorSubcoreMesh(core_axis_name="c", subcore_axis_name="s")

    def body(col_hbm, row_hbm, o_hbm, scol, srow, buf0, buf1, sem0, sem1):
        wid = lax.axis_index("s") * _NC + lax.axis_index("c")
        i0 = wid * ipw
        bufs, sems = (buf0, buf1), (sem0, sem1)

        start = pl.multiple_of(jnp.minimum(i0 - i0 % 8, h - 16), 8)
        off = i0 - start
        cp = pltpu.make_async_copy(row_hbm.at[pl.ds(start, 16)], srow, sem0)
        cp.start()
        cp.wait()

        for c in range(w // _CHUNK):
            cp = pltpu.make_async_copy(
                col_hbm.at[pl.ds(c * _CHUNK, _CHUNK)], scol, sem0
            )
            cp.start()
            cp.wait()

            def colrow(r, carry):
                j = c * _CHUNK + r
                for jj in range(nv):
                    v = scol.at[r][pl.ds(jj * _L, _L)]
                    buf0.at[j][pl.ds(jj * _L, _L)] = v
                    buf1.at[j][pl.ds(jj * _L, _L)] = v
                return carry

            lax.fori_loop(0, _CHUNK, colrow, 0, unroll=False)

        waits = {}
        for k in range(ipw):
            buf, sem = bufs[k % 2], sems[k % 2]
            if k >= 2:
                for cp in waits[k - 2]:
                    cp.wait()
            vs = [srow.at[off + k][pl.ds(jj * _L, _L)] for jj in range(nv)]

            def rowfill(j, carry):
                for jj in range(nv):
                    buf.at[j][pl.ds(d_half + jj * _L, _L)] = vs[jj]
                return carry

            lax.fori_loop(0, h, rowfill, 0, unroll=False)
            waits[k] = []
            for bb in range(b):
                cp = pltpu.make_async_copy(buf, o_hbm.at[bb, i0 + k], sem)
                cp.start()
                waits[k].append(cp)
        for k in (ipw - 2, ipw - 1):
            for cp in waits[k]:
                cp.wait()

    return pl.kernel(
        body,
        out_type=jax.ShapeDtypeStruct((b, h, w, d), jnp.float32),
        mesh=mesh,
        scratch_types=[
            pltpu.VMEM((_CHUNK, d_half), jnp.float32),
            pltpu.VMEM((16, d_half), jnp.float32),
            pltpu.VMEM((w, d), jnp.float32),
            pltpu.VMEM((w, d), jnp.float32),
            pltpu.SemaphoreType.DMA,
            pltpu.SemaphoreType.DMA,
        ],
        compiler_params=pltpu.CompilerParams(use_tc_tiling_on_sc=True),
    )


def kernel(x, row_embed, col_embed):
    b = x.shape[0]
    h, w = x.shape[2], x.shape[3]
    d_half = row_embed.shape[1]
    d = 2 * d_half
    out_bhwd = _sc_fill(b=b, d=d, h=h, w=w)(col_embed[:w], row_embed[:h])
    return jnp.transpose(out_bhwd, (0, 3, 1, 2)).astype(x.dtype)


# speedup vs baseline: 4.9265x; 1.0102x over previous
import jax
import jax.numpy as jnp
from jax import lax
from jax.experimental import pallas as pl
from jax.experimental.pallas import tpu as pltpu
from jax.experimental.pallas import tpu_sc as plsc

_NC, _NS, _L = 2, 16, 16
_CHUNK = 56


def _sc_fill(*, b, d, h, w):
    nw = _NC * _NS
    ipw = h // nw
    d_half = d // 2
    nv = d_half // _L
    mesh = plsc.VectorSubcoreMesh(core_axis_name="c", subcore_axis_name="s")

    def body(col_hbm, row_hbm, o_hbm, scol, srow, buf0, buf1, sem0, sem1):
        wid = lax.axis_index("s") * _NC + lax.axis_index("c")
        i0 = wid * ipw
        bufs, sems = (buf0, buf1), (sem0, sem1)

        start = pl.multiple_of(jnp.minimum(i0 - i0 % 8, h - 16), 8)
        off = i0 - start
        cp = pltpu.make_async_copy(row_hbm.at[pl.ds(start, 16)], srow, sem0)
        cp.start()
        cp.wait()

        cpa = pltpu.make_async_copy(
            col_hbm, buf0.at[:, pl.ds(0, d_half)], sem0
        )
        cpb = pltpu.make_async_copy(
            col_hbm, buf1.at[:, pl.ds(0, d_half)], sem1
        )
        cpa.start()
        cpb.start()
        cpa.wait()
        cpb.wait()

        waits = {}
        for k in range(ipw):
            buf, sem = bufs[k % 2], sems[k % 2]
            if k >= 2:
                for cp in waits[k - 2]:
                    cp.wait()
            vs = [srow.at[off + k][pl.ds(jj * _L, _L)] for jj in range(nv)]

            def rowfill(j, carry):
                for jj in range(nv):
                    buf.at[j][pl.ds(d_half + jj * _L, _L)] = vs[jj]
                return carry

            lax.fori_loop(0, h, rowfill, 0, unroll=False)
            waits[k] = []
            for bb in range(b):
                cp = pltpu.make_async_copy(buf, o_hbm.at[bb, i0 + k], sem)
                cp.start()
                waits[k].append(cp)
        for k in (ipw - 2, ipw - 1):
            for cp in waits[k]:
                cp.wait()

    return pl.kernel(
        body,
        out_type=jax.ShapeDtypeStruct((b, h, w, d), jnp.float32),
        mesh=mesh,
        scratch_types=[
            pltpu.VMEM((_CHUNK, d_half), jnp.float32),
            pltpu.VMEM((16, d_half), jnp.float32),
            pltpu.VMEM((w, d), jnp.float32),
            pltpu.VMEM((w, d), jnp.float32),
            pltpu.SemaphoreType.DMA,
            pltpu.SemaphoreType.DMA,
        ],
        compiler_params=pltpu.CompilerParams(use_tc_tiling_on_sc=True),
    )


def kernel(x, row_embed, col_embed):
    b = x.shape[0]
    h, w = x.shape[2], x.shape[3]
    d_half = row_embed.shape[1]
    d = 2 * d_half
    out_bhwd = _sc_fill(b=b, d=d, h=h, w=w)(col_embed[:w], row_embed[:h])
    return jnp.transpose(out_bhwd, (0, 3, 1, 2)).astype(x.dtype)
